# baseline (speedup 1.0000x reference)
import dataclasses
import functools

import jax
import jax.numpy as jnp
from jax import lax
from jax.experimental import pallas as pl
from jax.experimental.pallas import tpu as pltpu
from jax.experimental.pallas import tpu_sc as plsc

N = 10000
E = 320000
D = 128
C = 40

NC = 2
NS = 16
NW = NC * NS
EPW = E // NW
K = 80
PADC = EPW + 4 * K
NPAD = 10240
NR = 3
RNG = 3456
AROWS = RNG + 128
RPT = AROWS // NS
BN = 1000

_mesh = plsc.VectorSubcoreMesh(core_axis_name="c", subcore_axis_name="s")
_f32 = jnp.float32

_sc_params = pltpu.CompilerParams()
if "needs_layout_passes" in pltpu.CompilerParams.__dataclass_fields__:
    _sc_params = dataclasses.replace(_sc_params, needs_layout_passes=False)


@functools.partial(
    pl.kernel, mesh=_mesh, compiler_params=_sc_params,
    out_type=[jax.ShapeDtypeStruct((E,), _f32),
              jax.ShapeDtypeStruct((NW, N), _f32)],
    scratch_types=[pltpu.VMEM((EPW,), jnp.int32),
                   pltpu.VMEM((EPW,), jnp.int32),
                   pltpu.VMEM((N,), _f32),
                   pltpu.VMEM((N,), _f32),
                   pltpu.VMEM((EPW,), _f32),
                   pltpu.VMEM((N,), _f32)],
)
def _sc_edges(src_hbm, dst_hbm, s_hbm, d_hbm, ee_hbm, denp_hbm,
              src_v, dst_v, s_v, d_v, ee_v, den_v):
    wid = lax.axis_index("s") * NC + lax.axis_index("c")
    base = wid * EPW
    pltpu.sync_copy(src_hbm.at[pl.ds(base, EPW)], src_v)
    pltpu.sync_copy(dst_hbm.at[pl.ds(base, EPW)], dst_v)
    pltpu.sync_copy(s_hbm, s_v)
    pltpu.sync_copy(d_hbm, d_v)

    @pl.loop(0, N, step=16)
    def _(i):
        den_v[pl.ds(i, 16)] = jnp.zeros((16,), _f32)

    @pl.loop(0, EPW, step=16)
    def _(i):
        sidx = src_v[pl.ds(i, 16)]
        didx = dst_v[pl.ds(i, 16)]
        e = plsc.load_gather(s_v, [sidx]) + plsc.load_gather(d_v, [didx])
        e = jnp.where(e >= 0.0, e, 0.2 * e)
        ee = jnp.exp(e)
        ee_v[pl.ds(i, 16)] = ee
        plsc.addupdate_scatter(den_v, [didx], ee)

    pltpu.sync_copy(ee_v, ee_hbm.at[pl.ds(base, EPW)])
    pltpu.sync_copy(den_v, denp_hbm.at[wid])


def _make_sc_rows(rbase):
    @functools.partial(
        pl.kernel, mesh=_mesh, compiler_params=_sc_params,
        out_type=jax.ShapeDtypeStruct((NC, AROWS, D), _f32),
        scratch_types=[pltpu.VMEM((PADC,), jnp.int32),
                       pltpu.VMEM((PADC,), jnp.int32),
                       pltpu.VMEM((PADC,), _f32),
                       pltpu.VMEM((K,), jnp.int32),
                       pltpu.VMEM((K,), jnp.int32),
                       pltpu.VMEM((K, D), _f32),
                       pltpu.VMEM((K, D), _f32),
                       pltpu.VMEM((K, D), _f32),
                       pltpu.VMEM((K, D), _f32),
                       pltpu.VMEM((112, D), _f32),
                       pltpu.VMEM_SHARED((AROWS, D), _f32),
                       pltpu.SemaphoreType.DMA,
                       pltpu.SemaphoreType.DMA],
    )
    def _sc_rows(src_hbm, dst_hbm, ee_hbm, m_hbm, outp_hbm,
                 src_v, dst_v, ee_v, ix0, ix1, gb0, gb1, sb0, sb1,
                 zb_v, acc_sh, gs0, gs1):
        cid = lax.axis_index("c")
        sid = lax.axis_index("s")
        wid = sid * NC + cid
        base = wid * EPW
        pltpu.sync_copy(src_hbm.at[pl.ds(base, EPW)], src_v.at[pl.ds(0, EPW)])
        pltpu.sync_copy(dst_hbm.at[pl.ds(base, EPW)], dst_v.at[pl.ds(0, EPW)])
        pltpu.sync_copy(ee_hbm.at[pl.ds(base, EPW)], ee_v.at[pl.ds(0, EPW)])

        @plsc.parallel_loop(0, 112)
        def _(r):
            for cc in range(0, D, 16):
                zb_v[r, pl.ds(cc, 16)] = jnp.zeros((16,), _f32)

        pltpu.sync_copy(zb_v, acc_sh.at[pl.ds(sid * RPT, 112)])
        pltpu.sync_copy(zb_v, acc_sh.at[pl.ds(sid * RPT + 112, 112)])

        def _compact(i, off):
            b16 = i * 16
            loc = dst_v[pl.ds(b16, 16)] - rbase
            inr = jnp.logical_and(loc >= 0, loc < RNG)
            plsc.store_compressed(src_v.at[pl.ds(off, 16)],
                                  src_v[pl.ds(b16, 16)], mask=inr)
            plsc.store_compressed(dst_v.at[pl.ds(off, 16)], loc, mask=inr)
            plsc.store_compressed(ee_v.at[pl.ds(off, 16)],
                                  ee_v[pl.ds(b16, 16)], mask=inr)
            return off + jnp.sum(inr.astype(jnp.int32), axis=0)

        cnt = lax.fori_loop(0, EPW // 16, _compact, jnp.int32(0))

        for c in range(0, 4 * K, 16):
            src_v[pl.ds(cnt + c, 16)] = jnp.zeros((16,), jnp.int32)
            dst_v[pl.ds(cnt + c, 16)] = jnp.full((16,), RNG, jnp.int32)
            ee_v[pl.ds(cnt + c, 16)] = jnp.zeros((16,), _f32)

        nch2 = 2 * ((cnt + (2 * K - 1)) // (2 * K))

        plsc.subcore_barrier()

        def _g_start(ch, gb, gs):
            pltpu.async_copy(m_hbm.at[src_v.at[pl.ds(ch * K, K)]], gb, gs)

        def _g_wait(gb, gs):
            pltpu.make_async_copy(m_hbm.at[src_v.at[pl.ds(0, K)]],
                                  gb, gs).wait()

        _g_start(0, gb0, gs0)
        _g_start(1, gb1, gs1)

        def _step(tt, b, ix, gb, sb, gs):
            a = 2 * tt + b
            cb = a * K
            _g_wait(gb, gs)

            for c in range(0, K, 16):
                ix[pl.ds(c, 16)] = dst_v[pl.ds(cb + c, 16)]

            @plsc.parallel_loop(0, K, unroll=4)
            def _(j):
                bc = plsc.load_gather(
                    ee_v, [jnp.full((16,), cb + j, jnp.int32)])
                for cc in range(0, D, 16):
                    sb[j, pl.ds(cc, 16)] = gb[j, pl.ds(cc, 16)] * bc

            _g_start(a + 2, gb, gs)
            pltpu.sync_copy(sb, acc_sh.at[ix], add=True)

        def _pair(tt, carry):
            _step(tt, 0, ix0, gb0, sb0, gs0)
            _step(tt, 1, ix1, gb1, sb1, gs1)
            return carry

        lax.fori_loop(0, nch2 // 2, _pair, jnp.int32(0))

        _g_wait(gb0, gs0)
        _g_wait(gb1, gs1)

        plsc.subcore_barrier()

        pltpu.sync_copy(acc_sh.at[pl.ds(sid * RPT, RPT)],
                        outp_hbm.at[cid, pl.ds(sid * RPT, RPT)])

    return _sc_rows


_sc_rows_r = [_make_sc_rows(r * RNG) for r in range(NR)]


@functools.partial(
    pl.kernel, mesh=_mesh, compiler_params=_sc_params,
    out_type=jax.ShapeDtypeStruct((E,), _f32),
    scratch_types=[pltpu.VMEM((EPW,), jnp.int32),
                   pltpu.VMEM((N,), _f32),
                   pltpu.VMEM((EPW,), _f32),
                   pltpu.VMEM((EPW,), _f32)],
)
def _sc_alpha(dst_hbm, ee_hbm, rden_hbm, alpha_hbm, dst_v, rden_v, ee_v, a_v):
    wid = lax.axis_index("s") * NC + lax.axis_index("c")
    base = wid * EPW
    pltpu.sync_copy(dst_hbm.at[pl.ds(base, EPW)], dst_v)
    pltpu.sync_copy(ee_hbm.at[pl.ds(base, EPW)], ee_v)
    pltpu.sync_copy(rden_hbm, rden_v)

    @pl.loop(0, EPW, step=16)
    def _(i):
        didx = dst_v[pl.ds(i, 16)]
        a_v[pl.ds(i, 16)] = ee_v[pl.ds(i, 16)] * plsc.load_gather(rden_v, [didx])

    pltpu.sync_copy(a_v, alpha_hbm.at[pl.ds(base, EPW)])


def _tc_pre(h, W):
    def body(h_ref, w_ref, m_ref):
        m_ref[...] = jnp.dot(h_ref[...], w_ref[...],
                             preferred_element_type=_f32)

    return pl.pallas_call(
        body,
        grid=(N // BN,),
        in_specs=[pl.BlockSpec((BN, D), lambda i: (i, 0)),
                  pl.BlockSpec((D, D), lambda i: (0, 0))],
        out_specs=pl.BlockSpec((BN, D), lambda i: (i, 0)),
        out_shape=jax.ShapeDtypeStruct((N, D), _f32),
    )(h, W)


def _tc_sd(m, a_s, a_d):
    def body(m_ref, as_ref, ad_ref, s_ref, d_ref):
        mm = m_ref[...]
        s_ref[...] = jnp.sum(mm * as_ref[...][None, :], axis=1)
        d_ref[...] = jnp.sum(mm * ad_ref[...][None, :], axis=1)

    return pl.pallas_call(
        body,
        out_shape=[jax.ShapeDtypeStruct((N,), _f32),
                   jax.ShapeDtypeStruct((N,), _f32)],
    )(m, a_s, a_d)


def _tc_rden(denp3):
    def body(p_ref, r_ref):
        r_ref[...] = 1.0 / (jnp.sum(p_ref[...], axis=0) + 1e-16)

    return pl.pallas_call(
        body,
        out_shape=jax.ShapeDtypeStruct((N // BN, BN), _f32),
    )(denp3)


def _tc_combine(outp, rden2, h_prev, V, Wn):
    def body(o_ref, r_ref, hp_ref, v_ref, wn_ref, h1_ref, err_ref, m1_ref):
        r = r_ref[0, 0][:, None]
        z = (o_ref[0] + o_ref[1]) * r
        h1 = jnp.where(z > 0.0, z, jnp.exp(z) - 1.0)
        h1_ref[...] = h1
        err_ref[...] = hp_ref[...] - jnp.dot(h1, v_ref[...],
                                             preferred_element_type=_f32)
        m1_ref[...] = jnp.dot(h1, wn_ref[...], preferred_element_type=_f32)

    return pl.pallas_call(
        body,
        grid=(N // BN,),
        in_specs=[pl.BlockSpec((NC, BN, D), lambda i: (0, i, 0)),
                  pl.BlockSpec((1, 1, BN), lambda i: (i, 0, 0)),
                  pl.BlockSpec((BN, D), lambda i: (i, 0)),
                  pl.BlockSpec((D, D), lambda i: (0, 0)),
                  pl.BlockSpec((D, D), lambda i: (0, 0))],
        out_specs=[pl.BlockSpec((BN, D), lambda i: (i, 0)),
                   pl.BlockSpec((BN, D), lambda i: (i, 0)),
                   pl.BlockSpec((BN, D), lambda i: (i, 0))],
        out_shape=[jax.ShapeDtypeStruct((N, D), _f32),
                   jax.ShapeDtypeStruct((N, D), _f32),
                   jax.ShapeDtypeStruct((N, D), _f32)],
    )(outp, rden2, h_prev, V, Wn)


def _tc_final(outp, rden2, h1, V, Wh, bh):
    def body(o_ref, r_ref, h1_ref, v_ref, wh_ref, bh_ref, err_ref, lg_ref):
        r = r_ref[0, 0][:, None]
        z = (o_ref[0] + o_ref[1]) * r
        h2 = jnp.where(z > 0.0, z, jnp.exp(z) - 1.0)
        err_ref[...] = h1_ref[...] - jnp.dot(h2, v_ref[...],
                                             preferred_element_type=_f32)
        lg_ref[...] = (jnp.dot(h2, wh_ref[...], preferred_element_type=_f32)
                       + bh_ref[...][None, :])

    return pl.pallas_call(
        body,
        grid=(N // BN,),
        in_specs=[pl.BlockSpec((NC, BN, D), lambda i: (0, i, 0)),
                  pl.BlockSpec((1, 1, BN), lambda i: (i, 0, 0)),
                  pl.BlockSpec((BN, D), lambda i: (i, 0)),
                  pl.BlockSpec((D, D), lambda i: (0, 0)),
                  pl.BlockSpec((D, C), lambda i: (0, 0)),
                  pl.BlockSpec((C,), lambda i: (0,))],
        out_specs=[pl.BlockSpec((BN, D), lambda i: (i, 0)),
                   pl.BlockSpec((BN, C), lambda i: (i, 0))],
        out_shape=[jax.ShapeDtypeStruct((N, D), _f32),
                   jax.ShapeDtypeStruct((N, C), _f32)],
    )(outp, rden2, h1, V, Wh, bh)


def kernel(x, edge_index, W0, a_src0, a_dst0, V0, W1, a_src1, a_dst1, V1,
           W_head, b_head):
    src = edge_index[0]
    dst = edge_index[1]

    def _rows_all(ee, m):
        parts = [rk(src, dst, ee, m) for rk in _sc_rows_r]
        return jnp.concatenate(
            [p[:, :min(RNG, NPAD - r * RNG)] for r, p in enumerate(parts)],
            axis=1)

    m0 = _tc_pre(x, W0)
    s0, d0 = _tc_sd(m0, a_src0, a_dst0)
    ee0, denp0 = _sc_edges(src, dst, s0, d0)
    rden0 = _tc_rden(denp0.reshape(NW, N // BN, BN))
    outp0 = _rows_all(ee0, m0)
    alpha0 = _sc_alpha(dst, ee0, rden0.reshape(N))
    h1, err0, m1 = _tc_combine(outp0, rden0.reshape(N // BN, 1, BN), x, V0, W1)
    s1, d1 = _tc_sd(m1, a_src1, a_dst1)
    ee1, denp1 = _sc_edges(src, dst, s1, d1)
    rden1 = _tc_rden(denp1.reshape(NW, N // BN, BN))
    outp1 = _rows_all(ee1, m1)
    alpha1 = _sc_alpha(dst, ee1, rden1.reshape(N))
    err1, logits = _tc_final(outp1, rden1.reshape(N // BN, 1, BN), h1,
                             V1, W_head, b_head)

    return (logits, (err0, err1), (alpha0, alpha1))


# speedup vs baseline: 2.1333x; 2.1333x over previous
import dataclasses
import functools

import jax
import jax.numpy as jnp
from jax import lax
from jax.experimental import pallas as pl
from jax.experimental.pallas import tpu as pltpu
from jax.experimental.pallas import tpu_sc as plsc

N = 10000
E = 320000
D = 128
C = 40

NC = 2
NS = 16
NW = NC * NS
EPW = E // NW
K = 80
PADC = EPW + 4 * K
NPAD = 10240
NR = 3
RNG = 3456
AROWS = RNG + 128
RPT = AROWS // NS
BN = 1000

_mesh = plsc.VectorSubcoreMesh(core_axis_name="c", subcore_axis_name="s")
_f32 = jnp.float32

_sc_params = pltpu.CompilerParams()
if "needs_layout_passes" in pltpu.CompilerParams.__dataclass_fields__:
    _sc_params = dataclasses.replace(_sc_params, needs_layout_passes=False)


@functools.partial(
    pl.kernel, mesh=_mesh, compiler_params=_sc_params,
    out_type=[jax.ShapeDtypeStruct((E,), _f32),
              jax.ShapeDtypeStruct((NW, N), _f32)],
    scratch_types=[pltpu.VMEM((EPW,), jnp.int32),
                   pltpu.VMEM((EPW,), jnp.int32),
                   pltpu.VMEM((N,), _f32),
                   pltpu.VMEM((N,), _f32),
                   pltpu.VMEM((EPW,), _f32),
                   pltpu.VMEM((N,), _f32)],
)
def _sc_edges(src_hbm, dst_hbm, s_hbm, d_hbm, ee_hbm, denp_hbm,
              src_v, dst_v, s_v, d_v, ee_v, den_v):
    wid = lax.axis_index("s") * NC + lax.axis_index("c")
    base = wid * EPW
    pltpu.sync_copy(src_hbm.at[pl.ds(base, EPW)], src_v)
    pltpu.sync_copy(dst_hbm.at[pl.ds(base, EPW)], dst_v)
    pltpu.sync_copy(s_hbm, s_v)
    pltpu.sync_copy(d_hbm, d_v)

    @pl.loop(0, N, step=16)
    def _(i):
        den_v[pl.ds(i, 16)] = jnp.zeros((16,), _f32)

    @pl.loop(0, EPW, step=16)
    def _(i):
        sidx = src_v[pl.ds(i, 16)]
        didx = dst_v[pl.ds(i, 16)]
        e = plsc.load_gather(s_v, [sidx]) + plsc.load_gather(d_v, [didx])
        e = jnp.where(e >= 0.0, e, 0.2 * e)
        ee = jnp.exp(e)
        ee_v[pl.ds(i, 16)] = ee
        plsc.addupdate_scatter(den_v, [didx], ee)

    pltpu.sync_copy(ee_v, ee_hbm.at[pl.ds(base, EPW)])
    pltpu.sync_copy(den_v, denp_hbm.at[wid])


def _make_sc_rows(rbase):
    @functools.partial(
        pl.kernel, mesh=_mesh, compiler_params=_sc_params,
        out_type=jax.ShapeDtypeStruct((NC, AROWS, D), _f32),
        scratch_types=[pltpu.VMEM((PADC,), jnp.int32),
                       pltpu.VMEM((PADC,), jnp.int32),
                       pltpu.VMEM((PADC,), _f32),
                       pltpu.VMEM((K,), jnp.int32),
                       pltpu.VMEM((K,), jnp.int32),
                       pltpu.VMEM((K, D), _f32),
                       pltpu.VMEM((K, D), _f32),
                       pltpu.VMEM((K, D), _f32),
                       pltpu.VMEM((K, D), _f32),
                       pltpu.VMEM((112, D), _f32),
                       pltpu.VMEM_SHARED((AROWS, D), _f32),
                       pltpu.SemaphoreType.DMA,
                       pltpu.SemaphoreType.DMA],
    )
    def _sc_rows(src_hbm, dst_hbm, ee_hbm, m_hbm, outp_hbm,
                 src_v, dst_v, ee_v, ix0, ix1, gb0, gb1, sb0, sb1,
                 zb_v, acc_sh, gs0, gs1):
        cid = lax.axis_index("c")
        sid = lax.axis_index("s")
        wid = sid * NC + cid
        base = wid * EPW
        pltpu.sync_copy(src_hbm.at[pl.ds(base, EPW)], src_v.at[pl.ds(0, EPW)])
        pltpu.sync_copy(dst_hbm.at[pl.ds(base, EPW)], dst_v.at[pl.ds(0, EPW)])
        pltpu.sync_copy(ee_hbm.at[pl.ds(base, EPW)], ee_v.at[pl.ds(0, EPW)])

        @plsc.parallel_loop(0, 112)
        def _(r):
            for cc in range(0, D, 16):
                zb_v[r, pl.ds(cc, 16)] = jnp.zeros((16,), _f32)

        pltpu.sync_copy(zb_v, acc_sh.at[pl.ds(sid * RPT, 112)])
        pltpu.sync_copy(zb_v, acc_sh.at[pl.ds(sid * RPT + 112, 112)])

        def _compact(i, off):
            b16 = i * 16
            loc = dst_v[pl.ds(b16, 16)] - rbase
            inr = jnp.logical_and(loc >= 0, loc < RNG)
            plsc.store_compressed(src_v.at[pl.ds(off, 16)],
                                  src_v[pl.ds(b16, 16)], mask=inr)
            plsc.store_compressed(dst_v.at[pl.ds(off, 16)], loc, mask=inr)
            plsc.store_compressed(ee_v.at[pl.ds(off, 16)],
                                  ee_v[pl.ds(b16, 16)], mask=inr)
            return off + jnp.sum(inr.astype(jnp.int32), axis=0)

        cnt = lax.fori_loop(0, EPW // 16, _compact, jnp.int32(0))

        for c in range(0, 4 * K, 16):
            src_v[pl.ds(cnt + c, 16)] = jnp.zeros((16,), jnp.int32)
            dst_v[pl.ds(cnt + c, 16)] = jnp.full((16,), RNG, jnp.int32)
            ee_v[pl.ds(cnt + c, 16)] = jnp.zeros((16,), _f32)

        nch = (cnt + (K - 1)) // K

        plsc.subcore_barrier()

        def _chunk(t, carry):
            cb = t * K
            for c in range(0, K, 16):
                ix0[pl.ds(c, 16)] = dst_v[pl.ds(cb + c, 16)]
            pltpu.sync_copy(m_hbm.at[src_v.at[pl.ds(cb, K)]], gb0)

            @plsc.parallel_loop(0, K, unroll=4)
            def _(j):
                bc = plsc.load_gather(
                    ee_v, [jnp.full((16,), cb + j, jnp.int32)])
                for cc in range(0, D, 16):
                    gb0[j, pl.ds(cc, 16)] = gb0[j, pl.ds(cc, 16)] * bc

            pltpu.sync_copy(gb0, acc_sh.at[ix0], add=True)
            return carry

        lax.fori_loop(0, nch, _chunk, jnp.int32(0))

        plsc.subcore_barrier()

        pltpu.sync_copy(acc_sh.at[pl.ds(sid * RPT, RPT)],
                        outp_hbm.at[cid, pl.ds(sid * RPT, RPT)])

    return _sc_rows


_sc_rows_r = [_make_sc_rows(r * RNG) for r in range(NR)]


@functools.partial(
    pl.kernel, mesh=_mesh, compiler_params=_sc_params,
    out_type=jax.ShapeDtypeStruct((E,), _f32),
    scratch_types=[pltpu.VMEM((EPW,), jnp.int32),
                   pltpu.VMEM((N,), _f32),
                   pltpu.VMEM((EPW,), _f32),
                   pltpu.VMEM((EPW,), _f32)],
)
def _sc_alpha(dst_hbm, ee_hbm, rden_hbm, alpha_hbm, dst_v, rden_v, ee_v, a_v):
    wid = lax.axis_index("s") * NC + lax.axis_index("c")
    base = wid * EPW
    pltpu.sync_copy(dst_hbm.at[pl.ds(base, EPW)], dst_v)
    pltpu.sync_copy(ee_hbm.at[pl.ds(base, EPW)], ee_v)
    pltpu.sync_copy(rden_hbm, rden_v)

    @pl.loop(0, EPW, step=16)
    def _(i):
        didx = dst_v[pl.ds(i, 16)]
        a_v[pl.ds(i, 16)] = ee_v[pl.ds(i, 16)] * plsc.load_gather(rden_v, [didx])

    pltpu.sync_copy(a_v, alpha_hbm.at[pl.ds(base, EPW)])


def _tc_pre(h, W):
    def body(h_ref, w_ref, m_ref):
        m_ref[...] = jnp.dot(h_ref[...], w_ref[...],
                             preferred_element_type=_f32)

    return pl.pallas_call(
        body,
        grid=(N // BN,),
        in_specs=[pl.BlockSpec((BN, D), lambda i: (i, 0)),
                  pl.BlockSpec((D, D), lambda i: (0, 0))],
        out_specs=pl.BlockSpec((BN, D), lambda i: (i, 0)),
        out_shape=jax.ShapeDtypeStruct((N, D), _f32),
    )(h, W)


def _tc_sd(m, a_s, a_d):
    def body(m_ref, as_ref, ad_ref, s_ref, d_ref):
        mm = m_ref[...]
        s_ref[...] = jnp.sum(mm * as_ref[...][None, :], axis=1)
        d_ref[...] = jnp.sum(mm * ad_ref[...][None, :], axis=1)

    return pl.pallas_call(
        body,
        out_shape=[jax.ShapeDtypeStruct((N,), _f32),
                   jax.ShapeDtypeStruct((N,), _f32)],
    )(m, a_s, a_d)


def _tc_rden(denp3):
    def body(p_ref, r_ref):
        r_ref[...] = 1.0 / (jnp.sum(p_ref[...], axis=0) + 1e-16)

    return pl.pallas_call(
        body,
        out_shape=jax.ShapeDtypeStruct((N // BN, BN), _f32),
    )(denp3)


def _tc_combine(outp, rden2, h_prev, V, Wn):
    def body(o_ref, r_ref, hp_ref, v_ref, wn_ref, h1_ref, err_ref, m1_ref):
        r = r_ref[0, 0][:, None]
        z = (o_ref[0] + o_ref[1]) * r
        h1 = jnp.where(z > 0.0, z, jnp.exp(z) - 1.0)
        h1_ref[...] = h1
        err_ref[...] = hp_ref[...] - jnp.dot(h1, v_ref[...],
                                             preferred_element_type=_f32)
        m1_ref[...] = jnp.dot(h1, wn_ref[...], preferred_element_type=_f32)

    return pl.pallas_call(
        body,
        grid=(N // BN,),
        in_specs=[pl.BlockSpec((NC, BN, D), lambda i: (0, i, 0)),
                  pl.BlockSpec((1, 1, BN), lambda i: (i, 0, 0)),
                  pl.BlockSpec((BN, D), lambda i: (i, 0)),
                  pl.BlockSpec((D, D), lambda i: (0, 0)),
                  pl.BlockSpec((D, D), lambda i: (0, 0))],
        out_specs=[pl.BlockSpec((BN, D), lambda i: (i, 0)),
                   pl.BlockSpec((BN, D), lambda i: (i, 0)),
                   pl.BlockSpec((BN, D), lambda i: (i, 0))],
        out_shape=[jax.ShapeDtypeStruct((N, D), _f32),
                   jax.ShapeDtypeStruct((N, D), _f32),
                   jax.ShapeDtypeStruct((N, D), _f32)],
    )(outp, rden2, h_prev, V, Wn)


def _tc_final(outp, rden2, h1, V, Wh, bh):
    def body(o_ref, r_ref, h1_ref, v_ref, wh_ref, bh_ref, err_ref, lg_ref):
        r = r_ref[0, 0][:, None]
        z = (o_ref[0] + o_ref[1]) * r
        h2 = jnp.where(z > 0.0, z, jnp.exp(z) - 1.0)
        err_ref[...] = h1_ref[...] - jnp.dot(h2, v_ref[...],
                                             preferred_element_type=_f32)
        lg_ref[...] = (jnp.dot(h2, wh_ref[...], preferred_element_type=_f32)
                       + bh_ref[...][None, :])

    return pl.pallas_call(
        body,
        grid=(N // BN,),
        in_specs=[pl.BlockSpec((NC, BN, D), lambda i: (0, i, 0)),
                  pl.BlockSpec((1, 1, BN), lambda i: (i, 0, 0)),
                  pl.BlockSpec((BN, D), lambda i: (i, 0)),
                  pl.BlockSpec((D, D), lambda i: (0, 0)),
                  pl.BlockSpec((D, C), lambda i: (0, 0)),
                  pl.BlockSpec((C,), lambda i: (0,))],
        out_specs=[pl.BlockSpec((BN, D), lambda i: (i, 0)),
                   pl.BlockSpec((BN, C), lambda i: (i, 0))],
        out_shape=[jax.ShapeDtypeStruct((N, D), _f32),
                   jax.ShapeDtypeStruct((N, C), _f32)],
    )(outp, rden2, h1, V, Wh, bh)


def kernel(x, edge_index, W0, a_src0, a_dst0, V0, W1, a_src1, a_dst1, V1,
           W_head, b_head):
    src = edge_index[0]
    dst = edge_index[1]

    def _rows_all(ee, m):
        parts = [rk(src, dst, ee, m) for rk in _sc_rows_r]
        return jnp.concatenate(
            [p[:, :min(RNG, NPAD - r * RNG)] for r, p in enumerate(parts)],
            axis=1)

    m0 = _tc_pre(x, W0)
    s0, d0 = _tc_sd(m0, a_src0, a_dst0)
    ee0, denp0 = _sc_edges(src, dst, s0, d0)
    rden0 = _tc_rden(denp0.reshape(NW, N // BN, BN))
    outp0 = _rows_all(ee0, m0)
    alpha0 = _sc_alpha(dst, ee0, rden0.reshape(N))
    h1, err0, m1 = _tc_combine(outp0, rden0.reshape(N // BN, 1, BN), x, V0, W1)
    s1, d1 = _tc_sd(m1, a_src1, a_dst1)
    ee1, denp1 = _sc_edges(src, dst, s1, d1)
    rden1 = _tc_rden(denp1.reshape(NW, N // BN, BN))
    outp1 = _rows_all(ee1, m1)
    alpha1 = _sc_alpha(dst, ee1, rden1.reshape(N))
    err1, logits = _tc_final(outp1, rden1.reshape(N // BN, 1, BN), h1,
                             V1, W_head, b_head)

    return (logits, (err0, err1), (alpha0, alpha1))
